# baseline (speedup 1.0000x reference)
import functools

import jax
import jax.numpy as jnp
from jax import lax
from jax.experimental import pallas as pl
from jax.experimental.pallas import tpu as pltpu
from jax.experimental.pallas import tpu_sc as plsc

N_NODES_ = 10000
D_ = 128
N_POOL_ = 5000
N_EDGES_ = 320000
NW_ = 32
NB_ = N_EDGES_ // 128

PP_ = 160
PB_ = N_POOL_ - PP_
BW_ = 79
BB_ = NB_ - BW_
BH_ = 40
CR_ = 320
CB_ = N_NODES_ - CR_
CH_ = 160


HP_ = PP_ // 2


def _unpool_body(feat_hbm, pool0_hbm, pool1_hbm, edge_hbm,
                 outf_hbm, src_hbm, dst_hbm,
                 idx0_v, idx1_v, rows0_v, rows1_v, c0_v, c1_v, cb0_v, cb1_v,
                 gsems, nsems, esems, ssem, fsems, isems):
    wid = lax.axis_index("s") * 2 + lax.axis_index("c")

    base_p = jnp.minimum(wid * PP_, PB_)
    base_b = jnp.minimum(wid * BW_, BB_)
    base_c = jnp.minimum(wid * CR_, CB_)

    icopy0 = pltpu.async_copy(pool0_hbm.at[pl.ds(base_p, PP_)], idx0_v, isems[0])
    icopy1 = pltpu.async_copy(pool1_hbm.at[pl.ds(base_p, PP_)], idx1_v, isems[1])
    ecopies = []
    for h, (off, sz) in enumerate(((0, BH_), (BH_, BW_ - BH_))):
        ecopies.append(pltpu.async_copy(
            edge_hbm.at[pl.ds(base_b + off, sz), 0, :],
            c0_v.at[pl.ds(off, sz)], esems[2 * h]))
        ecopies.append(pltpu.async_copy(
            edge_hbm.at[pl.ds(base_b + off, sz), 1, :],
            c1_v.at[pl.ds(off, sz)], esems[2 * h + 1]))
    fin0 = pltpu.async_copy(feat_hbm.at[pl.ds(base_c, CH_)], cb0_v, fsems[0])
    fin1 = pltpu.async_copy(feat_hbm.at[pl.ds(base_c + CH_, CH_)], cb1_v, fsems[1])
    icopy0.wait()
    icopy1.wait()
    g = []
    for h in range(2):
        g.append(pltpu.async_copy(
            feat_hbm.at[idx0_v.at[pl.ds(h * HP_, HP_)]],
            rows0_v.at[pl.ds(h * HP_, HP_)], gsems[2 * h]))
        g.append(pltpu.async_copy(
            feat_hbm.at[idx1_v.at[pl.ds(h * HP_, HP_)]],
            rows1_v.at[pl.ds(h * HP_, HP_)], gsems[2 * h + 1]))

    scats = []
    for h, (off, sz) in enumerate(((0, BH_), (BH_, BW_ - BH_))):
        ecopies[2 * h].wait()
        scats.append(pltpu.async_copy(
            c0_v.at[pl.ds(off, sz)], src_hbm.at[pl.ds(base_b + off, sz)], ssem))
        scats.append(pltpu.async_copy(
            c0_v.at[pl.ds(off, sz)], dst_hbm.at[pl.ds(NB_ + base_b + off, sz)], ssem))
        ecopies[2 * h + 1].wait()
        scats.append(pltpu.async_copy(
            c1_v.at[pl.ds(off, sz)], src_hbm.at[pl.ds(NB_ + base_b + off, sz)], ssem))
        scats.append(pltpu.async_copy(
            c1_v.at[pl.ds(off, sz)], dst_hbm.at[pl.ds(base_b + off, sz)], ssem))

    fin0.wait()
    fout0 = pltpu.async_copy(cb0_v, outf_hbm.at[pl.ds(base_c, CH_)], fsems[0])
    fin1.wait()
    fout1 = pltpu.async_copy(cb1_v, outf_hbm.at[pl.ds(base_c + CH_, CH_)], fsems[1])

    def avg_row(j, carry):
        for d in range(D_ // 16):
            a = rows0_v[j, pl.ds(16 * d, 16)]
            b = rows1_v[j, pl.ds(16 * d, 16)]
            rows0_v[j, pl.ds(16 * d, 16)] = 0.5 * (a + b)
        return carry

    ncopies = []
    for h in range(2):
        g[2 * h].wait()
        g[2 * h + 1].wait()
        lax.fori_loop(h * HP_, (h + 1) * HP_, avg_row, 0, unroll=2)
        ncopies.append(pltpu.async_copy(
            rows0_v.at[pl.ds(h * HP_, HP_)],
            outf_hbm.at[pl.ds(N_NODES_ + base_p + h * HP_, HP_)], nsems[h]))

    for s in scats:
        s.wait()
    fout0.wait()
    fout1.wait()
    for n in ncopies:
        n.wait()


_unpool_sc = functools.partial(
    pl.kernel,
    out_type=[
        jax.ShapeDtypeStruct((N_NODES_ + N_POOL_, D_), jnp.float32),
        jax.ShapeDtypeStruct((2 * NB_, 128), jnp.int32),
        jax.ShapeDtypeStruct((2 * NB_, 128), jnp.int32),
    ],
    mesh=plsc.VectorSubcoreMesh(core_axis_name="c", subcore_axis_name="s"),
    compiler_params=pltpu.CompilerParams(
        needs_layout_passes=False, use_tc_tiling_on_sc=False),
    scratch_types=[
        pltpu.VMEM((PP_,), jnp.int32),
        pltpu.VMEM((PP_,), jnp.int32),
        pltpu.VMEM((PP_, D_), jnp.float32),
        pltpu.VMEM((PP_, D_), jnp.float32),
        pltpu.VMEM((BW_, 128), jnp.int32),
        pltpu.VMEM((BW_, 128), jnp.int32),
        pltpu.VMEM((CH_, D_), jnp.float32),
        pltpu.VMEM((CH_, D_), jnp.float32),
        [pltpu.SemaphoreType.DMA for _ in range(4)],
        [pltpu.SemaphoreType.DMA for _ in range(2)],
        [pltpu.SemaphoreType.DMA for _ in range(4)],
        pltpu.SemaphoreType.DMA,
        [pltpu.SemaphoreType.DMA for _ in range(2)],
        [pltpu.SemaphoreType.DMA for _ in range(2)],
    ],
)(_unpool_body)


@jax.jit
def kernel(feat, pool_idx_, edge_idx_):
    pool_i32 = pool_idx_.astype(jnp.int32)
    edge_i32 = edge_idx_.astype(jnp.int32)
    pool0 = pool_i32[:, 0]
    pool1 = pool_i32[:, 1]
    edge3 = edge_i32.reshape(NB_, 128, 2).transpose(0, 2, 1)
    feat_out, src2d, dst2d = _unpool_sc(feat, pool0, pool1, edge3)
    return feat_out, src2d.reshape(2 * N_EDGES_), dst2d.reshape(2 * N_EDGES_)


# speedup vs baseline: 1.0150x; 1.0150x over previous
import functools

import jax
import jax.numpy as jnp
from jax import lax
from jax.experimental import pallas as pl
from jax.experimental.pallas import tpu as pltpu
from jax.experimental.pallas import tpu_sc as plsc

N_NODES_ = 10000
D_ = 128
N_POOL_ = 5000
N_EDGES_ = 320000
NW_ = 32
NB_ = N_EDGES_ // 128

PP_ = 160
PB_ = N_POOL_ - PP_
BW_ = 79
BB_ = NB_ - BW_
BH_ = 40
CR_ = 320
CB_ = N_NODES_ - CR_
CH_ = 160


HP_ = PP_ // 2


def _unpool_body(feat_hbm, pool0_hbm, pool1_hbm,
                 outf_hbm,
                 idx0_v, idx1_v, rows0_v, rows1_v, cb0_v, cb1_v,
                 gsems, nsems, fsems, isems):
    wid = lax.axis_index("s") * 2 + lax.axis_index("c")

    base_p = jnp.minimum(wid * PP_, PB_)
    base_c = jnp.minimum(wid * CR_, CB_)

    icopy0 = pltpu.async_copy(pool0_hbm.at[pl.ds(base_p, PP_)], idx0_v, isems[0])
    icopy1 = pltpu.async_copy(pool1_hbm.at[pl.ds(base_p, PP_)], idx1_v, isems[1])
    fin0 = pltpu.async_copy(feat_hbm.at[pl.ds(base_c, CH_)], cb0_v, fsems[0])
    fin1 = pltpu.async_copy(feat_hbm.at[pl.ds(base_c + CH_, CH_)], cb1_v, fsems[1])
    icopy0.wait()
    icopy1.wait()
    g = []
    for h in range(2):
        g.append(pltpu.async_copy(
            feat_hbm.at[idx0_v.at[pl.ds(h * HP_, HP_)]],
            rows0_v.at[pl.ds(h * HP_, HP_)], gsems[2 * h]))
        g.append(pltpu.async_copy(
            feat_hbm.at[idx1_v.at[pl.ds(h * HP_, HP_)]],
            rows1_v.at[pl.ds(h * HP_, HP_)], gsems[2 * h + 1]))

    fin0.wait()
    fout0 = pltpu.async_copy(cb0_v, outf_hbm.at[pl.ds(base_c, CH_)], fsems[0])
    fin1.wait()
    fout1 = pltpu.async_copy(cb1_v, outf_hbm.at[pl.ds(base_c + CH_, CH_)], fsems[1])

    def avg_row(j, carry):
        for d in range(D_ // 16):
            a = rows0_v[j, pl.ds(16 * d, 16)]
            b = rows1_v[j, pl.ds(16 * d, 16)]
            rows0_v[j, pl.ds(16 * d, 16)] = 0.5 * (a + b)
        return carry

    ncopies = []
    for h in range(2):
        g[2 * h].wait()
        g[2 * h + 1].wait()
        lax.fori_loop(h * HP_, (h + 1) * HP_, avg_row, 0, unroll=2)
        ncopies.append(pltpu.async_copy(
            rows0_v.at[pl.ds(h * HP_, HP_)],
            outf_hbm.at[pl.ds(N_NODES_ + base_p + h * HP_, HP_)], nsems[h]))

    fout0.wait()
    fout1.wait()
    for n in ncopies:
        n.wait()


_unpool_sc = functools.partial(
    pl.kernel,
    out_type=jax.ShapeDtypeStruct((N_NODES_ + N_POOL_, D_), jnp.float32),
    mesh=plsc.VectorSubcoreMesh(core_axis_name="c", subcore_axis_name="s"),
    compiler_params=pltpu.CompilerParams(
        needs_layout_passes=False, use_tc_tiling_on_sc=False),
    scratch_types=[
        pltpu.VMEM((PP_,), jnp.int32),
        pltpu.VMEM((PP_,), jnp.int32),
        pltpu.VMEM((PP_, D_), jnp.float32),
        pltpu.VMEM((PP_, D_), jnp.float32),
        pltpu.VMEM((CH_, D_), jnp.float32),
        pltpu.VMEM((CH_, D_), jnp.float32),
        [pltpu.SemaphoreType.DMA for _ in range(4)],
        [pltpu.SemaphoreType.DMA for _ in range(2)],
        [pltpu.SemaphoreType.DMA for _ in range(2)],
        [pltpu.SemaphoreType.DMA for _ in range(2)],
    ],
)(_unpool_body)


EB_ = 64


def _edge_body(e_ref, src_ref, dst_ref):
    x = e_ref[...]
    c0 = jnp.stack([x[:, k, :] for k in (0, 2, 4, 6)], axis=1).reshape(4 * EB_, 128)
    c1 = jnp.stack([x[:, k, :] for k in (1, 3, 5, 7)], axis=1).reshape(4 * EB_, 128)
    src_ref[0] = c0
    src_ref[1] = c1
    dst_ref[0] = c1
    dst_ref[1] = c0


_edge_tc = pl.pallas_call(
    _edge_body,
    grid=((625 + EB_ - 1) // EB_,),
    in_specs=[pl.BlockSpec((EB_, 8, 128), lambda i: (i, 0, 0))],
    out_specs=[
        pl.BlockSpec((2, 4 * EB_, 128), lambda i: (0, i, 0)),
        pl.BlockSpec((2, 4 * EB_, 128), lambda i: (0, i, 0)),
    ],
    out_shape=[
        jax.ShapeDtypeStruct((2, NB_, 128), jnp.int32),
        jax.ShapeDtypeStruct((2, NB_, 128), jnp.int32),
    ],
)


@jax.jit
def kernel(feat, pool_idx_, edge_idx_):
    pool_i32 = pool_idx_.astype(jnp.int32)
    edge_i32 = edge_idx_.astype(jnp.int32)
    pool0 = pool_i32[:, 0]
    pool1 = pool_i32[:, 1]
    edge8 = (edge_i32.reshape(625, 4, 128, 2)
             .transpose(0, 1, 3, 2).reshape(625, 8, 128))
    feat_out = _unpool_sc(feat, pool0, pool1)
    src2d, dst2d = _edge_tc(edge8)
    return feat_out, src2d.reshape(2 * N_EDGES_), dst2d.reshape(2 * N_EDGES_)
